# baseline (speedup 1.0000x reference)
import functools

import jax
import jax.numpy as jnp
from jax import lax
from jax.experimental import pallas as pl
from jax.experimental.pallas import tpu as pltpu
from jax.experimental.pallas import tpu_sc as plsc

_LANES = 16
_NC = 2
_NS = 16
_NW = _NC * _NS
_CHUNK = 16384


def _sc_body(k, n, x_hbm, s_hbm, out_hbm,
             xb0, xb1, ob0, ob1, sbuf, si0, si1, so0, so1):
    wid = lax.axis_index("s") * _NC + lax.axis_index("c")
    per_w = n // _NW
    base = wid * per_w

    pltpu.sync_copy(s_hbm, sbuf)
    sv = sbuf[0]
    svB = sbuf[2]
    t7 = sbuf[3]
    t3 = sbuf[1]
    zero = jnp.zeros((_LANES,), jnp.int32)
    wvecs = {w: jnp.full((_LANES,), w, jnp.int32) for w in (8, 4, 2, 1)}

    xbufs, obufs = (xb0, xb1), (ob0, ob1)
    sins, souts = (si0, si1), (so0, so1)
    nchunks = per_w // _CHUNK
    in_h = [None] * nchunks
    out_h = [None] * nchunks

    def start_in(c):
        off = base + c * _CHUNK
        return pltpu.async_copy(
            x_hbm.at[pl.ds(off, _CHUNK)], xbufs[c % 2], sins[c % 2])

    in_h[0] = start_in(0)
    for c in range(nchunks):
        if c + 1 < nchunks:
            in_h[c + 1] = start_in(c + 1)
        in_h[c].wait()
        if c >= 2:
            out_h[c - 2].wait()
        xbuf, obuf = xbufs[c % 2], obufs[c % 2]

        @plsc.parallel_loop(0, _CHUNK // _LANES, step=1, unroll=2)
        def vstep(v):
            xv = xbuf[pl.ds(v * _LANES, _LANES)]
            pos = jnp.where(t7 < xv, wvecs[8],
                            jnp.where(t3 < xv, wvecs[4], zero))
            for w, svw in ((2, svB), (1, sv)):
                t = svw.at[pos].get(mode="promise_in_bounds")
                pos = pos + jnp.where(t < xv, wvecs[w], zero)
            obuf[pl.ds(v * _LANES, _LANES)] = pos

        off = base + c * _CHUNK
        out_h[c] = pltpu.async_copy(
            obuf, out_hbm.at[pl.ds(off, _CHUNK)], souts[c % 2])

    out_h[nchunks - 2].wait()
    out_h[nchunks - 1].wait()


def kernel(x, sorted_sequence):
    n = x.shape[0]
    k = sorted_sequence.shape[0]
    inf = jnp.full((_LANES,), jnp.inf, sorted_sequence.dtype)
    t = jnp.concatenate([sorted_sequence, inf[: _LANES - k]])
    smat = jnp.stack([
        t,
        jnp.broadcast_to(t[3], (_LANES,)),
        jnp.concatenate([t[1:], inf[:1]]),
        jnp.broadcast_to(t[7], (_LANES,)),
    ])

    mesh = plsc.VectorSubcoreMesh(core_axis_name="c", subcore_axis_name="s")
    f = pl.kernel(
        functools.partial(_sc_body, k, n),
        out_type=jax.ShapeDtypeStruct((n,), jnp.int32),
        mesh=mesh,
        scratch_types=[
            pltpu.VMEM((_CHUNK,), jnp.float32),
            pltpu.VMEM((_CHUNK,), jnp.float32),
            pltpu.VMEM((_CHUNK,), jnp.int32),
            pltpu.VMEM((_CHUNK,), jnp.int32),
            pltpu.VMEM((4, _LANES), jnp.float32),
            pltpu.SemaphoreType.DMA,
            pltpu.SemaphoreType.DMA,
            pltpu.SemaphoreType.DMA,
            pltpu.SemaphoreType.DMA,
        ],
    )
    return f(x, smat)


# speedup vs baseline: 1.0141x; 1.0141x over previous
import functools

import jax
import jax.numpy as jnp
from jax import lax
from jax.experimental import pallas as pl
from jax.experimental.pallas import tpu as pltpu
from jax.experimental.pallas import tpu_sc as plsc

_LANES = 16
_NC = 2
_NS = 16
_NW = _NC * _NS
_CHUNK = 16384


def _sc_body(k, n, x_hbm, s_hbm, out_hbm,
             xb0, xb1, ob0, ob1, sbuf, si0, si1, so0, so1):
    wid = lax.axis_index("s") * _NC + lax.axis_index("c")
    per_w = n // _NW
    base = wid * per_w

    pltpu.sync_copy(s_hbm, sbuf)
    sv = sbuf[0]
    svB = sbuf[2]
    t7 = sbuf[3]
    t3 = sbuf[1]
    zero = jnp.zeros((_LANES,), jnp.int32)
    wvecs = {w: jnp.full((_LANES,), w, jnp.int32) for w in (8, 4, 2, 1)}

    nchunks = per_w // _CHUNK
    npairs = nchunks // 2

    def issue_in(off, buf, sem):
        pltpu.async_copy(x_hbm.at[pl.ds(off, _CHUNK)], buf, sem)

    def wait_in(buf, sem):
        pltpu.make_async_copy(x_hbm.at[pl.ds(0, _CHUNK)], buf, sem).wait()

    def issue_out(buf, off, sem):
        pltpu.async_copy(buf, out_hbm.at[pl.ds(off, _CHUNK)], sem)

    def wait_out(buf, sem):
        pltpu.make_async_copy(buf, out_hbm.at[pl.ds(0, _CHUNK)], sem).wait()

    def compute(xbuf, obuf):
        @plsc.parallel_loop(0, _CHUNK // _LANES, step=1, unroll=4)
        def vstep(v):
            xv = xbuf[pl.ds(v * _LANES, _LANES)]
            pos = jnp.where(t7 < xv, wvecs[8],
                            jnp.where(t3 < xv, wvecs[4], zero))
            for w, svw in ((2, svB), (1, sv)):
                t = svw.at[pos].get(mode="promise_in_bounds")
                pos = pos + jnp.where(t < xv, wvecs[w], zero)
            obuf[pl.ds(v * _LANES, _LANES)] = pos

    issue_in(base, xb0, si0)

    def pair(i, carry):
        off0 = base + 2 * i * _CHUNK
        off1 = off0 + _CHUNK
        issue_in(off1, xb1, si1)
        wait_in(xb0, si0)

        @pl.when(i >= 1)
        def _():
            wait_out(ob0, so0)

        compute(xb0, ob0)
        issue_out(ob0, off0, so0)

        @pl.when(i < npairs - 1)
        def _():
            issue_in(off0 + 2 * _CHUNK, xb0, si0)

        wait_in(xb1, si1)

        @pl.when(i >= 1)
        def _():
            wait_out(ob1, so1)

        compute(xb1, ob1)
        issue_out(ob1, off1, so1)
        return carry

    lax.fori_loop(0, npairs, pair, 0)
    wait_out(ob0, so0)
    wait_out(ob1, so1)


def kernel(x, sorted_sequence):
    n = x.shape[0]
    k = sorted_sequence.shape[0]
    inf = jnp.full((_LANES,), jnp.inf, sorted_sequence.dtype)
    t = jnp.concatenate([sorted_sequence, inf[: _LANES - k]])
    smat = jnp.stack([
        t,
        jnp.broadcast_to(t[3], (_LANES,)),
        jnp.concatenate([t[1:], inf[:1]]),
        jnp.broadcast_to(t[7], (_LANES,)),
    ])

    mesh = plsc.VectorSubcoreMesh(core_axis_name="c", subcore_axis_name="s")
    f = pl.kernel(
        functools.partial(_sc_body, k, n),
        out_type=jax.ShapeDtypeStruct((n,), jnp.int32),
        mesh=mesh,
        scratch_types=[
            pltpu.VMEM((_CHUNK,), jnp.float32),
            pltpu.VMEM((_CHUNK,), jnp.float32),
            pltpu.VMEM((_CHUNK,), jnp.int32),
            pltpu.VMEM((_CHUNK,), jnp.int32),
            pltpu.VMEM((4, _LANES), jnp.float32),
            pltpu.SemaphoreType.DMA,
            pltpu.SemaphoreType.DMA,
            pltpu.SemaphoreType.DMA,
            pltpu.SemaphoreType.DMA,
        ],
    )
    return f(x, smat)
